# baseline (speedup 1.0000x reference)
import functools

import jax
import jax.numpy as jnp
from jax import lax
from jax.experimental import pallas as pl
from jax.experimental.pallas import tpu as pltpu
from jax.experimental.pallas import tpu_sc as plsc

HID = 64
NC = 2
NS = 16
NW = NC * NS
CHUNK = 1024


def _gather_body(table_hbm, idx_hbm, out_hbm, idx_v, rows_v, sem):
    wid = lax.axis_index("s") * NC + lax.axis_index("c")
    n_per_w = idx_hbm.shape[0] // NW
    base = wid * n_per_w
    nchunks = n_per_w // CHUNK

    def body(i, carry):
        off = base + i * CHUNK
        pltpu.sync_copy(idx_hbm.at[pl.ds(off, CHUNK)], idx_v)
        pltpu.async_copy(table_hbm.at[idx_v], rows_v, sem).wait()
        pltpu.sync_copy(rows_v, out_hbm.at[pl.ds(off, CHUNK)])
        return carry

    lax.fori_loop(0, nchunks, body, 0)


def kernel(actions, table):
    B, T = actions.shape
    n = B * T
    flat = actions.reshape(n)
    mesh = plsc.VectorSubcoreMesh(core_axis_name="c", subcore_axis_name="s")
    gather = functools.partial(
        pl.kernel,
        mesh=mesh,
        out_type=jax.ShapeDtypeStruct((n, HID), jnp.float32),
        scratch_types=[
            pltpu.VMEM((CHUNK,), jnp.int32),
            pltpu.VMEM((CHUNK, HID), jnp.float32),
            pltpu.SemaphoreType.DMA,
        ],
        compiler_params=pltpu.CompilerParams(use_tc_tiling_on_sc=False),
    )(_gather_body)
    out = gather(table, flat)
    return out.reshape(B, T, HID)


# speedup vs baseline: 1.0157x; 1.0157x over previous
import functools

import jax
import jax.numpy as jnp
from jax import lax
from jax.experimental import pallas as pl
from jax.experimental.pallas import tpu as pltpu
from jax.experimental.pallas import tpu_sc as plsc

HID = 64
NC = 2
NS = 16
NW = NC * NS
CHUNK = 400
NBUF = 4


def _gather_body(table_hbm, idx_hbm, out_hbm, idx_v, rows_v, gsem, osem):
    wid = lax.axis_index("s") * NC + lax.axis_index("c")
    n_per_w = idx_hbm.shape[0] // NW
    base = wid * n_per_w
    nchunks = n_per_w // CHUNK
    ngroups = nchunks // NBUF

    pltpu.sync_copy(idx_hbm.at[pl.ds(base, n_per_w)], idx_v)

    def gather_desc(chunk, b):
        return pltpu.make_async_copy(
            table_hbm.at[idx_v.at[pl.ds(chunk * CHUNK, CHUNK)]],
            rows_v.at[b],
            gsem.at[b],
        )

    def out_desc(chunk, b):
        return pltpu.make_async_copy(
            rows_v.at[b],
            out_hbm.at[pl.ds(base + chunk * CHUNK, CHUNK)],
            osem.at[b],
        )

    for b in range(NBUF):
        gather_desc(b, b).start()

    def group(g, carry):
        for b in range(NBUF):
            i = g * NBUF + b
            gather_desc(i, b).wait()
            od = out_desc(i, b)
            od.start()
            nxt = i + NBUF

            @pl.when(nxt < nchunks)
            def _():
                od.wait()
                gather_desc(nxt, b).start()

        return carry

    lax.fori_loop(0, ngroups, group, 0)

    for b in range(NBUF):
        out_desc(nchunks - NBUF + b, b).wait()


def kernel(actions, table):
    B, T = actions.shape
    n = B * T
    n_per_w = n // NW
    flat = actions.reshape(n)
    mesh = plsc.VectorSubcoreMesh(core_axis_name="c", subcore_axis_name="s")
    gather = functools.partial(
        pl.kernel,
        mesh=mesh,
        out_type=jax.ShapeDtypeStruct((n, HID), jnp.float32),
        scratch_types=[
            pltpu.VMEM((n_per_w,), jnp.int32),
            pltpu.VMEM((NBUF, CHUNK, HID), jnp.float32),
            pltpu.SemaphoreType.DMA((NBUF,)),
            pltpu.SemaphoreType.DMA((NBUF,)),
        ],
        compiler_params=pltpu.CompilerParams(use_tc_tiling_on_sc=False),
    )(_gather_body)
    out = gather(table, flat)
    return out.reshape(B, T, HID)


# speedup vs baseline: 1.2433x; 1.2241x over previous
import functools

import jax
import jax.numpy as jnp
from jax import lax
from jax.experimental import pallas as pl
from jax.experimental.pallas import tpu as pltpu
from jax.experimental.pallas import tpu_sc as plsc

HID = 64
PADH = 128
NC = 2
NS = 16
NW = NC * NS
CHUNK = 200
NBUF = 4


def _gather_body(table_hbm, idx_hbm, out_hbm, idx_v, rows_v, gsem, osem):
    wid = lax.axis_index("s") * NC + lax.axis_index("c")
    n_per_w = idx_hbm.shape[0] // NW
    base = wid * n_per_w
    nchunks = n_per_w // CHUNK

    pltpu.sync_copy(idx_hbm.at[pl.ds(base, n_per_w)], idx_v)

    def gather_desc(chunk, b):
        return pltpu.make_async_copy(
            table_hbm.at[idx_v.at[pl.ds(chunk * CHUNK, CHUNK)]],
            rows_v.at[b],
            gsem.at[b],
        )

    def out_desc(chunk, b):
        return pltpu.make_async_copy(
            rows_v.at[b],
            out_hbm.at[pl.ds(base + chunk * CHUNK, CHUNK)],
            osem.at[b],
        )

    for b in range(NBUF):
        gather_desc(b, b).start()

    def group(g, carry):
        for b in range(NBUF):
            i = g * NBUF + b
            gather_desc(i, b).wait()
            od = out_desc(i, b)
            od.start()
            nxt = i + NBUF

            @pl.when(nxt < nchunks)
            def _():
                od.wait()
                gather_desc(nxt, b).start()

        return carry

    lax.fori_loop(0, nchunks // NBUF, group, 0)

    for b in range(NBUF):
        out_desc(nchunks - NBUF + b, b).wait()


def kernel(actions, table):
    B, T = actions.shape
    n = B * T
    n_per_w = n // NW
    flat = actions.reshape(n)
    padded = jnp.pad(table, ((0, 0), (0, PADH - HID)))
    mesh = plsc.VectorSubcoreMesh(core_axis_name="c", subcore_axis_name="s")
    gather = functools.partial(
        pl.kernel,
        mesh=mesh,
        out_type=jax.ShapeDtypeStruct((n, PADH), jnp.float32),
        scratch_types=[
            pltpu.VMEM((n_per_w,), jnp.int32),
            pltpu.VMEM((NBUF, CHUNK, PADH), jnp.float32),
            pltpu.SemaphoreType.DMA((NBUF,)),
            pltpu.SemaphoreType.DMA((NBUF,)),
        ],
    )(_gather_body)
    out = gather(padded, flat)
    return out[:, :HID].reshape(B, T, HID)
